# Initial kernel scaffold; baseline (speedup 1.0000x reference)
import jax
import jax.numpy as jnp
from jax.experimental import pallas as pl


def kernel(input_x, table):
    raise NotImplementedError("write your pallas kernel here")


# speedup vs baseline: 2.8224x; 2.8224x over previous
import functools

import jax
import jax.numpy as jnp
from jax import lax
from jax.experimental import pallas as pl
from jax.experimental.pallas import tpu as pltpu
from jax.experimental.pallas import tpu_sc as plsc

_E = 64
_B = 1024
_S = 200
_N = _B * _S

_NC = 2
_NS = 16
_NW = _NC * _NS
_PER_W = _N // _NW
_C = 400
_NCHUNK = _PER_W // _C
_G = 80
_NG = _C // _G


def _pe_table():
    pos = jnp.arange(_S, dtype=jnp.float32)[:, None]
    denom = 10000.0 ** ((jnp.arange(_E) // 2).astype(jnp.float32) / _E)[None, :]
    ang = pos / denom
    return jnp.where((jnp.arange(_E) % 2)[None, :] == 0, jnp.sin(ang), jnp.cos(ang))


def _sc_call(idx_flat, table, pe):
    mesh = plsc.VectorSubcoreMesh(core_axis_name="c", subcore_axis_name="s")

    @functools.partial(
        pl.kernel,
        mesh=mesh,
        out_type=jax.ShapeDtypeStruct((_N, _E), jnp.float32),
        compiler_params=pltpu.CompilerParams(use_tc_tiling_on_sc=False),
        scratch_types=[
            pltpu.VMEM((_C,), jnp.int32),
            pltpu.VMEM((_C, _E), jnp.float32),
            pltpu.VMEM((_S, _E), jnp.float32),
            pltpu.SemaphoreType.DMA,
        ],
    )
    def k(idx_hbm, table_hbm, pe_hbm, out_hbm, idx_v, rows_v, pe_v, sem):
        wid = lax.axis_index("s") * _NC + lax.axis_index("c")
        pltpu.sync_copy(pe_hbm, pe_v)
        wbase = wid * _PER_W

        def chunk_body(ci, carry):
            base = wbase + ci * _C
            pltpu.sync_copy(idx_hbm.at[pl.ds(base, _C)], idx_v)
            handles = []
            for g in range(_NG):
                handles.append(pltpu.async_copy(
                    table_hbm.at[idx_v.at[pl.ds(g * _G, _G)]],
                    rows_v.at[pl.ds(g * _G, _G), :],
                    sem,
                ))
            for h in handles:
                h.wait()

            def add_body(r, c2):
                for c in range(_E // 16):
                    sl = pl.ds(c * 16, 16)
                    p0 = pe_v[r, sl]
                    rows_v[r, sl] += p0
                    rows_v[r + _S, sl] += p0
                return c2

            lax.fori_loop(0, _S, add_body, 0)
            pltpu.sync_copy(rows_v, out_hbm.at[pl.ds(base, _C)])
            return carry

        lax.fori_loop(0, _NCHUNK, chunk_body, 0)

    return k(idx_flat, table, pe)


def kernel(input_x, table):
    idx = input_x.reshape(_N).astype(jnp.int32)
    pe = _pe_table()
    out = _sc_call(idx, table, pe)
    return out.reshape(_B, _S, _E)


# speedup vs baseline: 3.1884x; 1.1297x over previous
import functools

import jax
import jax.numpy as jnp
from jax import lax
from jax.experimental import pallas as pl
from jax.experimental.pallas import tpu as pltpu
from jax.experimental.pallas import tpu_sc as plsc

_E = 64
_B = 1024
_S = 200
_N = _B * _S

_NC = 2
_NS = 16
_NW = _NC * _NS
_PER_W = _N // _NW
_C = 400
_NCHUNK = _PER_W // _C
_G = 80
_NG = _C // _G


def _pe_table():
    pos = jnp.arange(_S, dtype=jnp.float32)[:, None]
    denom = 10000.0 ** ((jnp.arange(_E) // 2).astype(jnp.float32) / _E)[None, :]
    ang = pos / denom
    return jnp.where((jnp.arange(_E) % 2)[None, :] == 0, jnp.sin(ang), jnp.cos(ang))


def _sc_call(idx_flat, table, pe):
    mesh = plsc.VectorSubcoreMesh(core_axis_name="c", subcore_axis_name="s")

    @functools.partial(
        pl.kernel,
        mesh=mesh,
        out_type=jax.ShapeDtypeStruct((_N, _E), jnp.float32),
        compiler_params=pltpu.CompilerParams(use_tc_tiling_on_sc=False),
        scratch_types=[
            pltpu.VMEM((_PER_W,), jnp.int32),
            pltpu.VMEM((_C, _E), jnp.float32),
            pltpu.VMEM((_C, _E), jnp.float32),
            pltpu.VMEM((_S, _E), jnp.float32),
            pltpu.SemaphoreType.DMA,
            pltpu.SemaphoreType.DMA,
            pltpu.SemaphoreType.DMA,
            pltpu.SemaphoreType.DMA,
        ],
    )
    def k(idx_hbm, table_hbm, pe_hbm, out_hbm, idx_v, rows0, rows1, pe_v,
          gs0, gs1, os0, os1):
        wid = lax.axis_index("s") * _NC + lax.axis_index("c")
        wbase = wid * _PER_W
        pltpu.sync_copy(idx_hbm.at[pl.ds(wbase, _PER_W)], idx_v)
        pltpu.sync_copy(pe_hbm, pe_v)

        rows = (rows0, rows1)
        gsem = (gs0, gs1)
        osem = (os0, os1)
        gh = [None, None]
        oh = [None, None]

        def fire(ci):
            b = ci & 1
            hs = []
            for g in range(_NG):
                hs.append(pltpu.async_copy(
                    table_hbm.at[idx_v.at[pl.ds(ci * _C + g * _G, _G)]],
                    rows[b].at[pl.ds(g * _G, _G), :],
                    gsem[b],
                ))
            gh[b] = hs

        def process(ci):
            b = ci & 1
            for h in gh[b]:
                h.wait()

            def add_body(r, c2):
                for c in range(_E // 16):
                    sl = pl.ds(c * 16, 16)
                    p0 = pe_v[r, sl]
                    plsc.addupdate(rows[b].at[r, sl], p0)
                    plsc.addupdate(rows[b].at[r + _S, sl], p0)
                return c2

            lax.fori_loop(0, _S, add_body, 0)
            oh[b] = pltpu.async_copy(
                rows[b], out_hbm.at[pl.ds(wbase + ci * _C, _C)], osem[b])

        fire(0)
        for ci in range(1, _NCHUNK + 1):
            if ci < _NCHUNK:
                b = ci & 1
                if oh[b] is not None:
                    oh[b].wait()
                fire(ci)
            process(ci - 1)
        oh[0].wait()
        oh[1].wait()

    return k(idx_flat, table, pe)


def kernel(input_x, table):
    idx = input_x.reshape(_N).astype(jnp.int32)
    pe = _pe_table()
    out = _sc_call(idx, table, pe)
    return out.reshape(_B, _S, _E)
